# baseline (speedup 1.0000x reference)
import jax
import jax.numpy as jnp
from jax.experimental import pallas as pl
from jax.experimental.pallas import tpu as pltpu

N = 512
EMB = 64
HD = 64
MID = 64
BOT = HD

BI = 256
KX = 72

_W1 = 0
_W2 = 192
_WSE = 256
_BSE = 258
_B1 = 259
_B2 = 260
_NPK = 261

_C00 = (((0,), (1,)), ((), ()))


def _pool_body(cx_ref, cy_ref, nei_ref, lstm_blk_ref, lstm_ref, Wp_ref,
               out_ref, X0_s, X1_s, V_s, W2T_s, H_s):
    k = pl.program_id(0)
    W1r = Wp_ref[_W1:_W1 + EMB, :]

    @pl.when(k == 0)
    def _():
        lstmT = lstm_ref[...].T.astype(jnp.bfloat16)
        At = jax.lax.dot_general(W1r, Wp_ref[_WSE:_WSE + 2, :], _C00,
                                 preferred_element_type=jnp.float32)
        W1jT = Wp_ref[_W1 + EMB:_W1 + EMB + HD, :].T
        biasT = jax.lax.dot_general(
            W1r, Wp_ref[_BSE:_BSE + 1, :], _C00,
            preferred_element_type=jnp.float32) + Wp_ref[_B1:_B1 + 1, :].T
        for X in (X0_s, X1_s):
            X[2:2 + HD, :] = lstmT
            X[2 + HD:3 + HD, :] = jnp.ones((1, N), jnp.bfloat16)
            X[3 + HD:, :] = jnp.zeros((KX - 3 - HD, N), jnp.bfloat16)
        V_s[:, 0:2] = At.astype(jnp.bfloat16)
        V_s[:, 2:2 + HD] = W1jT.astype(jnp.bfloat16)
        V_s[:, 2 + HD:3 + HD] = biasT.astype(jnp.bfloat16)
        V_s[:, 3 + HD:] = jnp.zeros((MID, KX - 3 - HD), jnp.bfloat16)
        W2T_s[...] = Wp_ref[_W2:_W2 + MID, :].T.astype(jnp.bfloat16)

    Cib = jax.lax.dot_general(
        Wp_ref[_W1 + EMB + HD:_W1 + 3 * HD, :], lstm_blk_ref[...], _C00,
        preferred_element_type=jnp.float32)
    V = V_s[...]
    for il in range(BI):
        X = X0_s if il % 2 == 0 else X1_s
        X[0:1, :] = cx_ref[il:il + 1, :].astype(jnp.bfloat16)
        X[1:2, :] = cy_ref[il:il + 1, :].astype(jnp.bfloat16)
        preF = jnp.dot(V, X[...], preferred_element_type=jnp.float32)
        H_s[:, il * N:(il + 1) * N] = jnp.maximum(
            preF + Cib[:, il:il + 1], 0.0).astype(jnp.bfloat16)
    P = jnp.dot(W2T_s[...], H_s[...],
                preferred_element_type=jnp.float32)
    cols = []
    for il in range(BI):
        masked = jnp.where(nei_ref[il:il + 1, :] > 0,
                           P[:, il * N:(il + 1) * N], -1e30)
        cols.append(jnp.max(masked, axis=1, keepdims=True))
    poolT = jnp.concatenate(cols, axis=1)
    out_ref[...] = jnp.maximum(poolT.T + Wp_ref[_B2:_B2 + 1, :], 0.0)


def kernel(corr_index, nei_index, nei_num, lstm_state, curr_pos_abs,
           W_se, b_se, W1, b1, W2, b2):
    cx = corr_index[:, :, 0]
    cy = corr_index[:, :, 1]
    Wp = jnp.concatenate(
        [W1, W2, W_se, b_se[None, :], b1[None, :], b2[None, :]], axis=0)

    out = pl.pallas_call(
        _pool_body,
        grid=(N // BI,),
        in_specs=[
            pl.BlockSpec((BI, N), lambda k: (k, 0)),
            pl.BlockSpec((BI, N), lambda k: (k, 0)),
            pl.BlockSpec((BI, N), lambda k: (k, 0)),
            pl.BlockSpec((BI, HD), lambda k: (k, 0)),
            pl.BlockSpec((N, HD), lambda k: (0, 0)),
            pl.BlockSpec((_NPK, MID), lambda k: (0, 0)),
        ],
        out_specs=pl.BlockSpec((BI, BOT), lambda k: (k, 0)),
        out_shape=jax.ShapeDtypeStruct((N, BOT), jnp.float32),
        scratch_shapes=[pltpu.VMEM((KX, N), jnp.bfloat16),
                        pltpu.VMEM((KX, N), jnp.bfloat16),
                        pltpu.VMEM((MID, KX), jnp.bfloat16),
                        pltpu.VMEM((BOT, MID), jnp.bfloat16),
                        pltpu.VMEM((MID, BI * N), jnp.bfloat16)],
    )(cx, cy, nei_index, lstm_state, lstm_state, Wp)
    return out


# speedup vs baseline: 1.0493x; 1.0493x over previous
import jax
import jax.numpy as jnp
from jax.experimental import pallas as pl
from jax.experimental.pallas import tpu as pltpu

N = 512
EMB = 64
HD = 64
MID = 64
BOT = HD

BI = 128
KX = 72

_W1 = 0
_W2 = 192
_WSE = 256
_BSE = 258
_B1 = 259
_B2 = 260
_NPK = 261

_C00 = (((0,), (1,)), ((), ()))


def _pool_body(cx_ref, cy_ref, nei_ref, lstm_blk_ref, lstm_ref, W1_ref,
               W2_ref, Wse_ref, bse_ref, b1_ref, b2_ref,
               out_ref, X0_s, X1_s, V_s, W2T_s, H_s):
    k = pl.program_id(0)
    W1r = W1_ref[:EMB, :]

    @pl.when(k == 0)
    def _():
        lstmT = lstm_ref[...].T.astype(jnp.bfloat16)
        At = jax.lax.dot_general(W1r, Wse_ref[...], _C00,
                                 preferred_element_type=jnp.float32)
        W1jT = W1_ref[EMB:EMB + HD, :].T
        biasT = jax.lax.dot_general(
            W1r, bse_ref[...][None, :], _C00,
            preferred_element_type=jnp.float32) + b1_ref[...][None, :].T
        for X in (X0_s, X1_s):
            X[2:2 + HD, :] = lstmT
            X[2 + HD:3 + HD, :] = jnp.ones((1, N), jnp.bfloat16)
            X[3 + HD:, :] = jnp.zeros((KX - 3 - HD, N), jnp.bfloat16)
        V_s[:, 0:2] = At.astype(jnp.bfloat16)
        V_s[:, 2:2 + HD] = W1jT.astype(jnp.bfloat16)
        V_s[:, 2 + HD:3 + HD] = biasT.astype(jnp.bfloat16)
        V_s[:, 3 + HD:] = jnp.zeros((MID, KX - 3 - HD), jnp.bfloat16)
        W2T_s[...] = W2_ref[...].T.astype(jnp.bfloat16)

    Cib = jax.lax.dot_general(
        W1_ref[EMB + HD:, :], lstm_blk_ref[...], _C00,
        preferred_element_type=jnp.float32)
    V = V_s[...]
    for il in range(BI):
        X = X0_s if il % 2 == 0 else X1_s
        X[0:1, :] = cx_ref[il:il + 1, :].astype(jnp.bfloat16)
        X[1:2, :] = cy_ref[il:il + 1, :].astype(jnp.bfloat16)
        preF = jnp.dot(V, X[...], preferred_element_type=jnp.float32)
        H_s[:, il * N:(il + 1) * N] = jnp.maximum(
            preF + Cib[:, il:il + 1], 0.0).astype(jnp.bfloat16)
    P = jnp.dot(W2T_s[...], H_s[...],
                preferred_element_type=jnp.float32)
    cols = []
    for il in range(BI):
        masked = jnp.where(nei_ref[il:il + 1, :] > 0,
                           P[:, il * N:(il + 1) * N], -1e30)
        cols.append(jnp.max(masked, axis=1, keepdims=True))
    poolT = jnp.concatenate(cols, axis=1)
    out_ref[...] = jnp.maximum(poolT.T + b2_ref[...][None, :], 0.0)


def kernel(corr_index, nei_index, nei_num, lstm_state, curr_pos_abs,
           W_se, b_se, W1, b1, W2, b2):
    cx = corr_index[:, :, 0]
    cy = corr_index[:, :, 1]

    out = pl.pallas_call(
        _pool_body,
        grid=(N // BI,),
        in_specs=[
            pl.BlockSpec((BI, N), lambda k: (k, 0)),
            pl.BlockSpec((BI, N), lambda k: (k, 0)),
            pl.BlockSpec((BI, N), lambda k: (k, 0)),
            pl.BlockSpec((BI, HD), lambda k: (k, 0)),
            pl.BlockSpec((N, HD), lambda k: (0, 0)),
            pl.BlockSpec((3 * HD, MID), lambda k: (0, 0)),
            pl.BlockSpec((MID, BOT), lambda k: (0, 0)),
            pl.BlockSpec((2, EMB), lambda k: (0, 0)),
            pl.BlockSpec((EMB,), lambda k: (0,)),
            pl.BlockSpec((MID,), lambda k: (0,)),
            pl.BlockSpec((BOT,), lambda k: (0,)),
        ],
        out_specs=pl.BlockSpec((BI, BOT), lambda k: (k, 0)),
        out_shape=jax.ShapeDtypeStruct((N, BOT), jnp.float32),
        scratch_shapes=[pltpu.VMEM((KX, N), jnp.bfloat16),
                        pltpu.VMEM((KX, N), jnp.bfloat16),
                        pltpu.VMEM((MID, KX), jnp.bfloat16),
                        pltpu.VMEM((BOT, MID), jnp.bfloat16),
                        pltpu.VMEM((MID, BI * N), jnp.bfloat16)],
    )(cx, cy, nei_index, lstm_state, lstm_state, W1, W2, W_se, b_se, b1, b2)
    return out
